# baseline (speedup 1.0000x reference)
import functools
import jax
import jax.numpy as jnp
from jax import lax
from jax.experimental import pallas as pl
from jax.experimental.pallas import tpu as pltpu


def _vq_body(x_ref, w_in_t_ref, b_in_ref, et2_ref, embed_ref,
             w_out_t_ref, b_out_ref, out_ref):
    z = jnp.dot(x_ref[...], w_in_t_ref[...],
                preferred_element_type=jnp.float32) + b_in_ref[...]
    et2 = et2_ref[...]
    s2 = jnp.dot(z, et2, preferred_element_type=jnp.float32)
    e_sq = 0.25 * jnp.sum(et2 * et2, axis=0, keepdims=True)
    scores = s2 - e_sq
    idx = jnp.argmax(scores, axis=1).astype(jnp.int32)
    iota = lax.broadcasted_iota(jnp.int32, scores.shape, 1)
    onehot = jnp.where(iota == idx[:, None], 1.0, 0.0)
    q = jnp.dot(onehot, embed_ref[...],
                preferred_element_type=jnp.float32)
    out_ref[...] = jnp.dot(q, w_out_t_ref[...],
                           preferred_element_type=jnp.float32) + b_out_ref[...]


@jax.jit
def kernel(x, W_in, b_in, W_out, b_out, embed):
    b, t, dim = x.shape
    cdim, _ = W_in.shape
    k = embed.shape[0]
    bt = b * t
    flat = x.reshape(bt, dim)
    r = 4096
    grid = (bt // r,)

    out = pl.pallas_call(
        _vq_body,
        grid=grid,
        in_specs=[
            pl.BlockSpec((r, dim), lambda i: (i, 0)),
            pl.BlockSpec((dim, cdim), lambda i: (0, 0)),
            pl.BlockSpec((1, cdim), lambda i: (0, 0)),
            pl.BlockSpec((cdim, k), lambda i: (0, 0)),
            pl.BlockSpec((k, cdim), lambda i: (0, 0)),
            pl.BlockSpec((cdim, dim), lambda i: (0, 0)),
            pl.BlockSpec((1, dim), lambda i: (0, 0)),
        ],
        out_specs=pl.BlockSpec((r, dim), lambda i: (i, 0)),
        out_shape=jax.ShapeDtypeStruct((bt, dim), jnp.float32),
    )(flat, W_in.T, b_in.reshape(1, cdim), 2.0 * embed.T, embed,
      W_out.T, b_out.reshape(1, dim))
    return out.reshape(b, t, dim)
